# baseline (speedup 1.0000x reference)
import functools

import jax
import jax.numpy as jnp
from jax import lax
from jax.experimental import pallas as pl
from jax.experimental.pallas import tpu as pltpu
from jax.experimental.pallas import tpu_sc as plsc

N = 10000
E = 320000
D = 128
DE = 16
H = 128
G = 32

NC, NS = 2, 16
NW = NC * NS
CHUNK = 128
NSEG = 4
NCHUNKS = E // CHUNK
SEG_CHS = [250, 1000, 1000, 250]
SEG_CH_STARTS = [0, 250, 1250, 2250]
SEG_E = [c * CHUNK for c in SEG_CHS]
SEG_E_STARTS = [c * CHUNK for c in SEG_CH_STARTS]
N_PAD = 10240
NROW = N_PAD // 128

MLP_BLK = 3200
PROJ_BLK = 1000

_f32 = jnp.float32


def _proj_body(na_ref, wa_ref, wb_ref, pa_ref, pb_ref):
    x = na_ref[...]
    pa_ref[...] = lax.dot_general(x, wa_ref[...], (((1,), (0,)), ((), ())),
                                  preferred_element_type=_f32)
    pb_ref[...] = lax.dot_general(x, wb_ref[...], (((1,), (0,)), ((), ())),
                                  preferred_element_type=_f32)


def _node_proj(node_attrs, w1a, w1b):
    grid = N // PROJ_BLK
    return pl.pallas_call(
        _proj_body,
        grid=(grid,),
        in_specs=[
            pl.BlockSpec((PROJ_BLK, D), lambda i: (i, 0)),
            pl.BlockSpec((D, H), lambda i: (0, 0)),
            pl.BlockSpec((D, H), lambda i: (0, 0)),
        ],
        out_specs=[
            pl.BlockSpec((PROJ_BLK, H), lambda i: (i, 0)),
            pl.BlockSpec((PROJ_BLK, H), lambda i: (i, 0)),
        ],
        out_shape=[
            jax.ShapeDtypeStruct((N, H), _f32),
            jax.ShapeDtypeStruct((N, H), _f32),
        ],
    )(node_attrs, w1a, w1b)


def _gather_body(seg, pa_hbm, pb_hbm, ei_hbm, x_hbm,
                 i0, i1, a0, b0, o0, a1, b1, o1, s0, s1, t0, t1):
    cid = lax.axis_index("c")
    sid = lax.axis_index("s")
    wid = sid * NC + cid
    start = SEG_CH_STARTS[seg]
    r = wid - (start % NW)
    r = jnp.where(r < 0, r + NW, r)
    nch = lax.shift_right_logical(SEG_CHS[seg] - r + NW - 1, 5)

    def issue(ci, idx2, ra, rb, sg):
        gbase = (start + r + ci * NW) * CHUNK
        pltpu.sync_copy(ei_hbm.at[:, pl.ds(gbase, CHUNK)], idx2)
        pltpu.async_copy(pa_hbm.at[idx2.at[0]], ra, sg)
        pltpu.async_copy(pb_hbm.at[idx2.at[1]], rb, sg)

    def wait_gather(ra, rb, sg):
        pltpu.make_async_copy(pa_hbm.at[pl.ds(0, CHUNK)], ra, sg).wait()
        pltpu.make_async_copy(pb_hbm.at[pl.ds(0, CHUNK)], rb, sg).wait()

    def add_store(ci, ra, rb, ro, ss):
        def row_body(rr, c2):
            for c in range(H // 16):
                sl = pl.ds(c * 16, 16)
                ro[rr, sl] = ra[rr, sl] + rb[rr, sl]
            return c2
        lax.fori_loop(0, CHUNK, row_body, 0)
        lbase = (r + ci * NW) * CHUNK
        pltpu.async_copy(ro, x_hbm.at[pl.ds(lbase, CHUNK)], ss)

    def wait_store(ro, ss):
        pltpu.make_async_copy(ro, x_hbm.at[pl.ds(0, CHUNK)], ss).wait()

    issue(0, i0, a0, b0, s0)

    @pl.when(nch > 1)
    def _():
        issue(1, i1, a1, b1, s1)

    def body(i, carry):
        wait_gather(a0, b0, s0)

        @pl.when(i > 0)
        def _():
            wait_store(o0, t0)
        add_store(2 * i, a0, b0, o0, t0)

        @pl.when(2 * i + 2 < nch)
        def _():
            issue(2 * i + 2, i0, a0, b0, s0)
        @pl.when(2 * i + 1 < nch)
        def _():
            wait_gather(a1, b1, s1)

            @pl.when(i > 0)
            def _():
                wait_store(o1, t1)
            add_store(2 * i + 1, a1, b1, o1, t1)

            @pl.when(2 * i + 3 < nch)
            def _():
                issue(2 * i + 3, i1, a1, b1, s1)
        return carry

    half = lax.shift_right_logical(nch + 1, 1)
    lax.fori_loop(0, half, body, 0)
    wait_store(o0, t0)

    @pl.when(nch > 1)
    def _():
        wait_store(o1, t1)


def _edge_gather(seg, pa, pb, ei):
    mesh = plsc.VectorSubcoreMesh(core_axis_name="c", subcore_axis_name="s")
    f = functools.partial(
        pl.kernel,
        out_type=jax.ShapeDtypeStruct((SEG_E[seg], H), _f32),
        mesh=mesh,
        compiler_params=pltpu.CompilerParams(needs_layout_passes=False),
        scratch_types=(
            [pltpu.VMEM((2, CHUNK), jnp.int32)] * 2
            + [pltpu.VMEM((CHUNK, H), _f32)] * 6
            + [pltpu.SemaphoreType.DMA] * 4
        ),
        name=f"edge_gather_seg{seg}",
    )(functools.partial(_gather_body, seg))
    return f(pa, pb, ei)


def _mlp_body(x_ref, emb_ref, w1e_ref, b1_ref, w2_ref, b2_ref, w3_ref, b3_ref,
              out_ref):
    h = x_ref[...] + lax.dot_general(
        emb_ref[...], w1e_ref[...], (((0,), (0,)), ((), ())),
        preferred_element_type=_f32) + b1_ref[...]
    h = h * jax.nn.sigmoid(h)
    h = lax.dot_general(h, w2_ref[...], (((1,), (0,)), ((), ())),
                        preferred_element_type=_f32) + b2_ref[...]
    h = h * jax.nn.sigmoid(h)
    c = lax.dot_general(w3_ref[...], h, (((1,), (1,)), ((), ())),
                        preferred_element_type=_f32) + b3_ref[...]
    out_ref[...] = c.reshape(1, 1, MLP_BLK)


def _edge_mlp(seg, x, emb_t, w1e, b1, w2, b2, w3r, b3):
    grid = SEG_E[seg] // MLP_BLK
    off = SEG_E_STARTS[seg] // MLP_BLK
    return pl.pallas_call(
        _mlp_body,
        grid=(grid,),
        in_specs=[
            pl.BlockSpec((MLP_BLK, H), lambda i: (i, 0)),
            pl.BlockSpec((DE, MLP_BLK), lambda i: (0, off + i)),
            pl.BlockSpec((DE, H), lambda i: (0, 0)),
            pl.BlockSpec((1, H), lambda i: (0, 0)),
            pl.BlockSpec((H, H), lambda i: (0, 0)),
            pl.BlockSpec((1, H), lambda i: (0, 0)),
            pl.BlockSpec((1, H), lambda i: (0, 0)),
            pl.BlockSpec((1, 1), lambda i: (0, 0)),
        ],
        out_specs=pl.BlockSpec((1, 1, MLP_BLK), lambda i: (i, 0, 0)),
        out_shape=jax.ShapeDtypeStruct((grid, 1, MLP_BLK), _f32),
        name=f"edge_mlp_seg{seg}",
    )(x, emb_t, w1e, b1, w2, b2, w3r, b3)


SC_CH = 2000
PER_W = E // NW


def _scatter_body(c0_hbm, c1_hbm, c2_hbm, c3_hbm, ei_hbm, parts_hbm,
                  vals, idxs, nacc, sem):
    cid = lax.axis_index("c")
    sid = lax.axis_index("s")
    wid = sid * NC + cid
    zero16 = jnp.zeros((16,), _f32)

    def z_body(j, c2):
        nacc[pl.ds(j * 16, 16)] = zero16
        return c2
    lax.fori_loop(0, N_PAD // 16, z_body, 0)

    def body(ci, carry):
        gbase = wid * PER_W + ci * SC_CH
        for s, ch_hbm in enumerate((c0_hbm, c1_hbm, c2_hbm, c3_hbm)):
            st = SEG_E_STARTS[s]

            @pl.when((gbase >= st) & (gbase < st + SEG_E[s]))
            def _():
                pltpu.sync_copy(ch_hbm.at[pl.ds(gbase - st, SC_CH)], vals)
        pltpu.sync_copy(ei_hbm.at[pl.ds(gbase, SC_CH)], idxs)

        def inner(j, c2):
            sl = pl.ds(j * 16, 16)
            iv = idxs[sl]
            vv = vals[sl]
            plsc.addupdate_scatter(nacc, [iv], vv)
            return c2
        lax.fori_loop(0, SC_CH // 16, inner, 0)
        return carry

    lax.fori_loop(0, PER_W // SC_CH, body, 0)
    pltpu.sync_copy(nacc, parts_hbm.at[wid])


def _scatter(seg_charges, ei):
    mesh = plsc.VectorSubcoreMesh(core_axis_name="c", subcore_axis_name="s")
    f = functools.partial(
        pl.kernel,
        out_type=jax.ShapeDtypeStruct((NW, N_PAD), _f32),
        mesh=mesh,
        compiler_params=pltpu.CompilerParams(needs_layout_passes=False),
        scratch_types=[
            pltpu.VMEM((SC_CH,), _f32),
            pltpu.VMEM((SC_CH,), jnp.int32),
            pltpu.VMEM((N_PAD,), _f32),
            pltpu.SemaphoreType.DMA,
        ],
    )(_scatter_body)
    return f(*seg_charges, ei)


def _reduce_body(parts_ref, batch_ref, atom_ref, tot_ref):
    acc = jnp.zeros((NROW, 128), _f32)
    for w in range(NW):
        acc = acc + parts_ref[w]
    atom_ref[...] = acc
    b = batch_ref[...]
    row_iota = lax.broadcasted_iota(jnp.int32, (G, 1), 0)
    tot = jnp.zeros((G, 1), _f32)
    for g in range(G):
        s = jnp.sum(jnp.where(b == g, acc, 0.0))
        tot = tot + jnp.where(row_iota == g, s, 0.0)
    tot_ref[...] = tot


def _reduce(parts, batch2d):
    return pl.pallas_call(
        _reduce_body,
        out_shape=[
            jax.ShapeDtypeStruct((NROW, 128), _f32),
            jax.ShapeDtypeStruct((G, 1), _f32),
        ],
    )(parts, batch2d)


def kernel(node_attrs, edge_index, edge_embedding, edge_lengths, pos, batch,
           W1, b1, W2, b2, W3, b3):
    w1a = W1[:D]
    w1b = W1[D:2 * D]
    w1e = W1[2 * D:]
    emb_t = edge_embedding.T

    pa, pb = _node_proj(node_attrs, w1a, w1b)
    seg_charges = []
    for seg in range(NSEG):
        x_s = _edge_gather(seg, pa, pb, edge_index)
        c_s = _edge_mlp(seg, x_s, emb_t, w1e, b1.reshape(1, H), W2,
                        b2.reshape(1, H), W3.reshape(1, H), b3.reshape(1, 1))
        seg_charges.append(c_s.reshape(SEG_E[seg]))
    parts = _scatter(seg_charges, edge_index[0]).reshape(NW, NROW, 128)
    batch2d = jnp.pad(batch, (0, N_PAD - N)).reshape(NROW, 128)
    atom2d, total = _reduce(parts, batch2d)
    atomic = atom2d.reshape(N_PAD)[:N].reshape(N, 1)
    return atomic, total


# speedup vs baseline: 1.0347x; 1.0347x over previous
import functools

import jax
import jax.numpy as jnp
from jax import lax
from jax.experimental import pallas as pl
from jax.experimental.pallas import tpu as pltpu
from jax.experimental.pallas import tpu_sc as plsc

N = 10000
E = 320000
D = 128
DE = 16
H = 128
G = 32

NC, NS = 2, 16
NW = NC * NS
CHUNK = 128
NSEG = 4
NCHUNKS = E // CHUNK
SEG_CH = NCHUNKS // NSEG
E_SEG = SEG_CH * CHUNK
N_PAD = 10240
NROW = N_PAD // 128

MLP_BLK = 3200
PROJ_BLK = 1000

_f32 = jnp.float32


def _proj_body(na_ref, wa_ref, wb_ref, pa_ref, pb_ref):
    x = na_ref[...]
    pa_ref[...] = lax.dot_general(x, wa_ref[...], (((1,), (0,)), ((), ())),
                                  preferred_element_type=_f32)
    pb_ref[...] = lax.dot_general(x, wb_ref[...], (((1,), (0,)), ((), ())),
                                  preferred_element_type=_f32)


def _node_proj(node_attrs, w1a, w1b):
    grid = N // PROJ_BLK
    return pl.pallas_call(
        _proj_body,
        grid=(grid,),
        in_specs=[
            pl.BlockSpec((PROJ_BLK, D), lambda i: (i, 0)),
            pl.BlockSpec((D, H), lambda i: (0, 0)),
            pl.BlockSpec((D, H), lambda i: (0, 0)),
        ],
        out_specs=[
            pl.BlockSpec((PROJ_BLK, H), lambda i: (i, 0)),
            pl.BlockSpec((PROJ_BLK, H), lambda i: (i, 0)),
        ],
        out_shape=[
            jax.ShapeDtypeStruct((N, H), _f32),
            jax.ShapeDtypeStruct((N, H), _f32),
        ],
    )(node_attrs, w1a, w1b)


def _gather_body(seg, pa_hbm, pb_hbm, ei_hbm, x_hbm,
                 i0, i1, a0, b0, o0, a1, b1, o1, s0, s1, t0, t1):
    cid = lax.axis_index("c")
    sid = lax.axis_index("s")
    wid = sid * NC + cid
    start = seg * SEG_CH
    r = wid - (start % NW)
    r = jnp.where(r < 0, r + NW, r)
    nch = lax.shift_right_logical(SEG_CH - r + NW - 1, 5)

    def issue(ci, idx2, ra, rb, sg):
        gbase = (start + r + ci * NW) * CHUNK
        pltpu.sync_copy(ei_hbm.at[:, pl.ds(gbase, CHUNK)], idx2)
        pltpu.async_copy(pa_hbm.at[idx2.at[0]], ra, sg)
        pltpu.async_copy(pb_hbm.at[idx2.at[1]], rb, sg)

    def wait_gather(ra, rb, sg):
        pltpu.make_async_copy(pa_hbm.at[pl.ds(0, CHUNK)], ra, sg).wait()
        pltpu.make_async_copy(pb_hbm.at[pl.ds(0, CHUNK)], rb, sg).wait()

    def add_store(ci, ra, rb, ro, ss):
        def row_body(rr, c2):
            for c in range(H // 16):
                sl = pl.ds(c * 16, 16)
                ro[rr, sl] = ra[rr, sl] + rb[rr, sl]
            return c2
        lax.fori_loop(0, CHUNK, row_body, 0)
        lbase = (r + ci * NW) * CHUNK
        pltpu.async_copy(ro, x_hbm.at[pl.ds(lbase, CHUNK)], ss)

    def wait_store(ro, ss):
        pltpu.make_async_copy(ro, x_hbm.at[pl.ds(0, CHUNK)], ss).wait()

    issue(0, i0, a0, b0, s0)

    @pl.when(nch > 1)
    def _():
        issue(1, i1, a1, b1, s1)

    def body(i, carry):
        wait_gather(a0, b0, s0)

        @pl.when(i > 0)
        def _():
            wait_store(o0, t0)
        add_store(2 * i, a0, b0, o0, t0)

        @pl.when(2 * i + 2 < nch)
        def _():
            issue(2 * i + 2, i0, a0, b0, s0)
        @pl.when(2 * i + 1 < nch)
        def _():
            wait_gather(a1, b1, s1)

            @pl.when(i > 0)
            def _():
                wait_store(o1, t1)
            add_store(2 * i + 1, a1, b1, o1, t1)

            @pl.when(2 * i + 3 < nch)
            def _():
                issue(2 * i + 3, i1, a1, b1, s1)
        return carry

    half = lax.shift_right_logical(nch + 1, 1)
    lax.fori_loop(0, half, body, 0)
    wait_store(o0, t0)

    @pl.when(nch > 1)
    def _():
        wait_store(o1, t1)


def _edge_gather(seg, pa, pb, ei):
    mesh = plsc.VectorSubcoreMesh(core_axis_name="c", subcore_axis_name="s")
    f = functools.partial(
        pl.kernel,
        out_type=jax.ShapeDtypeStruct((E_SEG, H), _f32),
        mesh=mesh,
        compiler_params=pltpu.CompilerParams(needs_layout_passes=False),
        scratch_types=(
            [pltpu.VMEM((2, CHUNK), jnp.int32)] * 2
            + [pltpu.VMEM((CHUNK, H), _f32)] * 6
            + [pltpu.SemaphoreType.DMA] * 4
        ),
        name=f"edge_gather_seg{seg}",
    )(functools.partial(_gather_body, seg))
    return f(pa, pb, ei)


def _mlp_body(x_ref, emb_ref, w1e_ref, b1_ref, w2_ref, b2_ref, w3_ref, b3_ref,
              out_ref):
    h = x_ref[...] + lax.dot_general(
        emb_ref[...], w1e_ref[...], (((0,), (0,)), ((), ())),
        preferred_element_type=_f32) + b1_ref[...]
    h = h * jax.nn.sigmoid(h)
    h = lax.dot_general(h, w2_ref[...], (((1,), (0,)), ((), ())),
                        preferred_element_type=_f32) + b2_ref[...]
    h = h * jax.nn.sigmoid(h)
    c = lax.dot_general(w3_ref[...], h, (((1,), (1,)), ((), ())),
                        preferred_element_type=_f32) + b3_ref[...]
    out_ref[...] = c.reshape(1, 1, MLP_BLK)


def _edge_mlp(seg, x, emb_t, w1e, b1, w2, b2, w3r, b3):
    grid = E_SEG // MLP_BLK
    off = seg * grid
    return pl.pallas_call(
        _mlp_body,
        grid=(grid,),
        in_specs=[
            pl.BlockSpec((MLP_BLK, H), lambda i: (i, 0)),
            pl.BlockSpec((DE, MLP_BLK), lambda i: (0, off + i)),
            pl.BlockSpec((DE, H), lambda i: (0, 0)),
            pl.BlockSpec((1, H), lambda i: (0, 0)),
            pl.BlockSpec((H, H), lambda i: (0, 0)),
            pl.BlockSpec((1, H), lambda i: (0, 0)),
            pl.BlockSpec((1, H), lambda i: (0, 0)),
            pl.BlockSpec((1, 1), lambda i: (0, 0)),
        ],
        out_specs=pl.BlockSpec((1, 1, MLP_BLK), lambda i: (i, 0, 0)),
        out_shape=jax.ShapeDtypeStruct((grid, 1, MLP_BLK), _f32),
        name=f"edge_mlp_seg{seg}",
    )(x, emb_t, w1e, b1, w2, b2, w3r, b3)


SC_CH = 2000
PER_W = E // NW


def _scatter_body(c0_hbm, c1_hbm, c2_hbm, c3_hbm, ei_hbm, parts_hbm,
                  vals, idxs, nacc, sem):
    cid = lax.axis_index("c")
    sid = lax.axis_index("s")
    wid = sid * NC + cid
    zero16 = jnp.zeros((16,), _f32)

    def z_body(j, c2):
        nacc[pl.ds(j * 16, 16)] = zero16
        return c2
    lax.fori_loop(0, N_PAD // 16, z_body, 0)

    wps = NW // NSEG

    def body(ci, carry):
        gbase = wid * PER_W + ci * SC_CH
        lbase = lax.rem(wid, wps) * PER_W + ci * SC_CH
        for s, ch_hbm in enumerate((c0_hbm, c1_hbm, c2_hbm, c3_hbm)):
            @pl.when(lax.div(wid, wps) == s)
            def _():
                pltpu.sync_copy(ch_hbm.at[pl.ds(lbase, SC_CH)], vals)
        pltpu.sync_copy(ei_hbm.at[pl.ds(gbase, SC_CH)], idxs)

        def inner(j, c2):
            sl = pl.ds(j * 16, 16)
            iv = idxs[sl]
            vv = vals[sl]
            plsc.addupdate_scatter(nacc, [iv], vv)
            return c2
        lax.fori_loop(0, SC_CH // 16, inner, 0)
        return carry

    lax.fori_loop(0, PER_W // SC_CH, body, 0)
    pltpu.sync_copy(nacc, parts_hbm.at[wid])


def _scatter(seg_charges, ei):
    mesh = plsc.VectorSubcoreMesh(core_axis_name="c", subcore_axis_name="s")
    f = functools.partial(
        pl.kernel,
        out_type=jax.ShapeDtypeStruct((NW, N_PAD), _f32),
        mesh=mesh,
        compiler_params=pltpu.CompilerParams(needs_layout_passes=False),
        scratch_types=[
            pltpu.VMEM((SC_CH,), _f32),
            pltpu.VMEM((SC_CH,), jnp.int32),
            pltpu.VMEM((N_PAD,), _f32),
            pltpu.SemaphoreType.DMA,
        ],
    )(_scatter_body)
    return f(*seg_charges, ei)


def _reduce_body(parts_ref, batch_ref, atom_ref, tot_ref):
    acc = jnp.zeros((NROW, 128), _f32)
    for w in range(NW):
        acc = acc + parts_ref[w]
    atom_ref[...] = acc
    b = batch_ref[...]
    row_iota = lax.broadcasted_iota(jnp.int32, (G, 1), 0)
    tot = jnp.zeros((G, 1), _f32)
    for g in range(G):
        s = jnp.sum(jnp.where(b == g, acc, 0.0))
        tot = tot + jnp.where(row_iota == g, s, 0.0)
    tot_ref[...] = tot


def _reduce(parts, batch2d):
    return pl.pallas_call(
        _reduce_body,
        out_shape=[
            jax.ShapeDtypeStruct((NROW, 128), _f32),
            jax.ShapeDtypeStruct((G, 1), _f32),
        ],
    )(parts, batch2d)


def kernel(node_attrs, edge_index, edge_embedding, edge_lengths, pos, batch,
           W1, b1, W2, b2, W3, b3):
    w1a = W1[:D]
    w1b = W1[D:2 * D]
    w1e = W1[2 * D:]
    emb_t = edge_embedding.T

    pa, pb = _node_proj(node_attrs, w1a, w1b)
    seg_charges = []
    for seg in range(NSEG):
        x_s = _edge_gather(seg, pa, pb, edge_index)
        c_s = _edge_mlp(seg, x_s, emb_t, w1e, b1.reshape(1, H), W2,
                        b2.reshape(1, H), W3.reshape(1, H), b3.reshape(1, 1))
        seg_charges.append(c_s.reshape(E_SEG))
    parts = _scatter(seg_charges, edge_index[0]).reshape(NW, NROW, 128)
    batch2d = jnp.pad(batch, (0, N_PAD - N)).reshape(NROW, 128)
    atom2d, total = _reduce(parts, batch2d)
    atomic = atom2d.reshape(N_PAD)[:N].reshape(N, 1)
    return atomic, total
